# baseline (speedup 1.0000x reference)
import functools

import jax
import jax.numpy as jnp
from jax.experimental import pallas as pl
from jax.experimental.pallas import tpu as pltpu

_EPS = 1e-5
_BM = 320


def _norm_rows(z, f):
    zr = jnp.maximum(z, 0.0)
    mean = jnp.sum(zr, axis=1, keepdims=True) * (1.0 / f)
    diff = zr - mean
    var = jnp.sum(diff * diff, axis=1, keepdims=True) * (1.0 / max(f - 1, 1))
    return diff * pl.reciprocal(jnp.sqrt(var) + _EPS, approx=True)


def _body(x_ref, a_ref, w0_ref, b0_ref, w1_ref, b1_ref, w2_ref, b2_ref,
          out_ref, pen_ref,
          a32, abf, hb, hf32, xw, xv, wv0, wv, bv, outv,
          sem_a, sem_s, sem_o,
          *, n, f_in, fh, fo, nb):
    def slab(i):
        return pl.ds(i * _BM, _BM)

    a_cps = [pltpu.make_async_copy(a_ref.at[slab(i)], a32.at[slab(i)],
                                   sem_a.at[i]) for i in range(nb)]
    for cp in a_cps:
        cp.start()

    cp_x = pltpu.make_async_copy(x_ref, xv, sem_s.at[0])
    cp_w = pltpu.make_async_copy(w0_ref, wv0, sem_s.at[1])
    cp_b = pltpu.make_async_copy(b0_ref, bv, sem_s.at[2])
    cp_x.start()
    cp_w.start()
    cp_b.start()
    cp_x.wait()
    cp_w.wait()
    cp_b.wait()

    xw[...] = jnp.dot(xv[...], wv0[...],
                      preferred_element_type=jnp.float32).astype(jnp.bfloat16)

    a_cps[0].wait()
    abf[slab(0)] = a32[slab(0)].astype(jnp.bfloat16)
    z_prev = jnp.dot(abf[slab(0)], xw[...],
                     preferred_element_type=jnp.float32) + bv[...]
    for i in range(1, nb):
        a_cps[i].wait()
        abf[slab(i)] = a32[slab(i)].astype(jnp.bfloat16)
        z = jnp.dot(abf[slab(i)], xw[...],
                    preferred_element_type=jnp.float32) + bv[...]
        hb[slab(i - 1)] = _norm_rows(z_prev, fh).astype(jnp.bfloat16)
        z_prev = z
    hb[slab(nb - 1)] = _norm_rows(z_prev, fh).astype(jnp.bfloat16)

    cp_w = pltpu.make_async_copy(w1_ref, wv, sem_s.at[1])
    cp_b = pltpu.make_async_copy(b1_ref, bv, sem_s.at[2])
    cp_w.start()
    cp_b.start()
    cp_w.wait()
    cp_b.wait()
    xw[...] = jnp.dot(hb[...], wv[...],
                      preferred_element_type=jnp.float32).astype(jnp.bfloat16)
    z_prev = jnp.dot(abf[slab(0)], xw[...],
                     preferred_element_type=jnp.float32) + bv[...]
    for i in range(1, nb):
        z = jnp.dot(abf[slab(i)], xw[...],
                    preferred_element_type=jnp.float32) + bv[...]
        h2 = _norm_rows(z_prev, fh)
        hf32[slab(i - 1)] = h2
        hb[slab(i - 1)] = h2.astype(jnp.bfloat16)
        z_prev = z
    h2 = _norm_rows(z_prev, fh)
    hf32[slab(nb - 1)] = h2
    hb[slab(nb - 1)] = h2.astype(jnp.bfloat16)
    cp_pen = pltpu.make_async_copy(hf32, pen_ref, sem_o.at[0])
    cp_pen.start()

    cp_w = pltpu.make_async_copy(w2_ref, wv, sem_s.at[1])
    cp_b = pltpu.make_async_copy(b2_ref, bv, sem_s.at[2])
    cp_w.start()
    cp_b.start()
    cp_w.wait()
    cp_b.wait()
    xw[...] = jnp.dot(hb[...], wv[...],
                      preferred_element_type=jnp.float32).astype(jnp.bfloat16)
    z_prev = jnp.dot(abf[slab(0)], xw[...],
                     preferred_element_type=jnp.float32) + bv[...]
    for i in range(1, nb):
        z = jnp.dot(abf[slab(i)], xw[...],
                    preferred_element_type=jnp.float32) + bv[...]
        outv[slab(i - 1)] = z_prev[:, :fo]
        z_prev = z
    outv[slab(nb - 1)] = z_prev[:, :fo]
    cp_out = pltpu.make_async_copy(outv, out_ref, sem_o.at[1])
    cp_out.start()
    cp_pen.wait()
    cp_out.wait()


def kernel(x, a_hat, W0, b0, W1, b1, W2, b2):
    n, f_in = x.shape
    fh = W0.shape[1]
    fo = W2.shape[1]
    nb = n // _BM

    x_bf = x.astype(jnp.bfloat16)
    w0 = W0.astype(jnp.bfloat16)
    w1 = W1.astype(jnp.bfloat16)
    w2 = jnp.zeros((fh, fh), jnp.float32).at[:, :fo].set(W2).astype(
        jnp.bfloat16)
    b2p = jnp.zeros((1, fh), jnp.float32).at[:, :fo].set(b2.reshape(1, -1))

    body = functools.partial(_body, n=n, f_in=f_in, fh=fh, fo=fo, nb=nb)
    flops = 3 * 2 * n * n * fh + 2 * n * (f_in + 2 * fh) * fh
    out, pen = pl.pallas_call(
        body,
        out_shape=(jax.ShapeDtypeStruct((n, fo), jnp.float32),
                   jax.ShapeDtypeStruct((n, fh), jnp.float32)),
        in_specs=[pl.BlockSpec(memory_space=pl.ANY)] * 8,
        out_specs=(pl.BlockSpec(memory_space=pl.ANY),
                   pl.BlockSpec(memory_space=pl.ANY)),
        scratch_shapes=[
            pltpu.VMEM((n, n), jnp.float32),
            pltpu.VMEM((n, n), jnp.bfloat16),
            pltpu.VMEM((n, fh), jnp.bfloat16),
            pltpu.VMEM((n, fh), jnp.float32),
            pltpu.VMEM((n, fh), jnp.bfloat16),
            pltpu.VMEM((n, f_in), jnp.bfloat16),
            pltpu.VMEM((f_in, fh), jnp.bfloat16),
            pltpu.VMEM((fh, fh), jnp.bfloat16),
            pltpu.VMEM((1, fh), jnp.float32),
            pltpu.VMEM((n, fo), jnp.float32),
            pltpu.SemaphoreType.DMA((nb,)),
            pltpu.SemaphoreType.DMA((3,)),
            pltpu.SemaphoreType.DMA((2,)),
        ],
        compiler_params=pltpu.CompilerParams(
            vmem_limit_bytes=58 * 1024 * 1024,
        ),
        cost_estimate=pl.CostEstimate(
            flops=flops,
            transcendentals=2 * n,
            bytes_accessed=4 * (n * n + 4 * n * fh),
        ),
    )(x_bf, a_hat, w0, b0.reshape(1, -1), w1, b1.reshape(1, -1), w2, b2p)
    return out, pen


# speedup vs baseline: 1.3079x; 1.3079x over previous
import functools

import jax
import jax.numpy as jnp
from jax.experimental import pallas as pl
from jax.experimental.pallas import tpu as pltpu

_EPS = 1e-5
_BM = 320


def _norm_rows(z, f):
    zr = jnp.maximum(z, 0.0)
    mean = jnp.sum(zr, axis=1, keepdims=True) * (1.0 / f)
    diff = zr - mean
    var = jnp.sum(diff * diff, axis=1, keepdims=True) * (1.0 / max(f - 1, 1))
    return diff * pl.reciprocal(jnp.sqrt(var) + _EPS, approx=True)


def _body(x_ref, a_ref, w0_ref, b0_ref, w1_ref, b1_ref, w2_ref, b2_ref,
          out_ref, pen_ref,
          a32, xv, wv, bv, xwf, hf, outv, sem_a, sem_s, sem_o,
          *, n, f_in, fh, fo, nb):
    def slab(i):
        return pl.ds(i * _BM, _BM)

    a_cps = [pltpu.make_async_copy(a_ref.at[slab(i)], a32.at[slab(i)],
                                   sem_a.at[i]) for i in range(nb)]
    for cp in a_cps:
        cp.start()

    cp_x = pltpu.make_async_copy(x_ref, xv, sem_s.at[0])
    cp_w = pltpu.make_async_copy(w0_ref, wv.at[:f_in], sem_s.at[1])
    cp_b = pltpu.make_async_copy(b0_ref, bv, sem_s.at[2])
    cp_x.start()
    cp_w.start()
    cp_b.start()
    cp_x.wait()
    cp_w.wait()
    cp_b.wait()

    xwf[...] = jnp.dot(xv[...], wv[:f_in],
                       preferred_element_type=jnp.float32)

    a_cps[0].wait()
    z_prev = jnp.dot(a32[slab(0)], xwf[...],
                     preferred_element_type=jnp.float32) + bv[...]
    for i in range(1, nb):
        a_cps[i].wait()
        z = jnp.dot(a32[slab(i)], xwf[...],
                    preferred_element_type=jnp.float32) + bv[...]
        hf[slab(i - 1)] = _norm_rows(z_prev, fh)
        z_prev = z
    hf[slab(nb - 1)] = _norm_rows(z_prev, fh)

    cp_w = pltpu.make_async_copy(w1_ref, wv, sem_s.at[1])
    cp_b = pltpu.make_async_copy(b1_ref, bv, sem_s.at[2])
    cp_w.start()
    cp_b.start()
    cp_w.wait()
    cp_b.wait()
    xwf[...] = jnp.dot(hf[...], wv[...], preferred_element_type=jnp.float32)
    z_prev = jnp.dot(a32[slab(0)], xwf[...],
                     preferred_element_type=jnp.float32) + bv[...]
    for i in range(1, nb):
        z = jnp.dot(a32[slab(i)], xwf[...],
                    preferred_element_type=jnp.float32) + bv[...]
        hf[slab(i - 1)] = _norm_rows(z_prev, fh)
        z_prev = z
    hf[slab(nb - 1)] = _norm_rows(z_prev, fh)
    cp_pen = pltpu.make_async_copy(hf, pen_ref, sem_o.at[0])
    cp_pen.start()

    cp_w = pltpu.make_async_copy(w2_ref, wv, sem_s.at[1])
    cp_b = pltpu.make_async_copy(b2_ref, bv, sem_s.at[2])
    cp_w.start()
    cp_b.start()
    cp_w.wait()
    cp_b.wait()
    xwf[...] = jnp.dot(hf[...], wv[...], preferred_element_type=jnp.float32)
    z_prev = jnp.dot(a32[slab(0)], xwf[...],
                     preferred_element_type=jnp.float32) + bv[...]
    for i in range(1, nb):
        z = jnp.dot(a32[slab(i)], xwf[...],
                    preferred_element_type=jnp.float32) + bv[...]
        outv[slab(i - 1)] = z_prev[:, :fo]
        z_prev = z
    outv[slab(nb - 1)] = z_prev[:, :fo]
    cp_out = pltpu.make_async_copy(outv, out_ref, sem_o.at[1])
    cp_out.start()
    cp_pen.wait()
    cp_out.wait()


def kernel(x, a_hat, W0, b0, W1, b1, W2, b2):
    n, f_in = x.shape
    fh = W0.shape[1]
    fo = W2.shape[1]
    nb = n // _BM

    w2 = jnp.zeros((fh, fh), jnp.float32).at[:, :fo].set(W2)
    b2p = jnp.zeros((1, fh), jnp.float32).at[:, :fo].set(b2.reshape(1, -1))

    body = functools.partial(_body, n=n, f_in=f_in, fh=fh, fo=fo, nb=nb)
    flops = 3 * 2 * n * n * fh + 2 * n * (f_in + 2 * fh) * fh
    out, pen = pl.pallas_call(
        body,
        out_shape=(jax.ShapeDtypeStruct((n, fo), jnp.float32),
                   jax.ShapeDtypeStruct((n, fh), jnp.float32)),
        in_specs=[pl.BlockSpec(memory_space=pl.ANY)] * 8,
        out_specs=(pl.BlockSpec(memory_space=pl.ANY),
                   pl.BlockSpec(memory_space=pl.ANY)),
        scratch_shapes=[
            pltpu.VMEM((n, n), jnp.float32),
            pltpu.VMEM((n, f_in), jnp.float32),
            pltpu.VMEM((fh, fh), jnp.float32),
            pltpu.VMEM((1, fh), jnp.float32),
            pltpu.VMEM((n, fh), jnp.float32),
            pltpu.VMEM((n, fh), jnp.float32),
            pltpu.VMEM((n, fo), jnp.float32),
            pltpu.SemaphoreType.DMA((nb,)),
            pltpu.SemaphoreType.DMA((3,)),
            pltpu.SemaphoreType.DMA((2,)),
        ],
        compiler_params=pltpu.CompilerParams(
            vmem_limit_bytes=52 * 1024 * 1024,
        ),
        cost_estimate=pl.CostEstimate(
            flops=flops,
            transcendentals=2 * n,
            bytes_accessed=4 * (n * n + 4 * n * fh),
        ),
    )(x, a_hat, W0, b0.reshape(1, -1), W1, b1.reshape(1, -1), w2, b2p)
    return out, pen


# speedup vs baseline: 1.3115x; 1.0028x over previous
import functools

import jax
import jax.numpy as jnp
from jax.experimental import pallas as pl
from jax.experimental.pallas import tpu as pltpu

_EPS = 1e-5
_BM = 320


def _norm_rows(z, f):
    zr = jnp.maximum(z, 0.0)
    mean = jnp.sum(zr, axis=1, keepdims=True) * (1.0 / f)
    diff = zr - mean
    var = jnp.sum(diff * diff, axis=1, keepdims=True) * (1.0 / max(f - 1, 1))
    return diff * pl.reciprocal(jnp.sqrt(var) + _EPS, approx=True)


def _body(x_ref, a_ref, w0_ref, b0_ref, w1_ref, b1_ref, w2_ref, b2_ref,
          out_ref, pen_ref,
          a32, xv, wv0, wv1, wv2, bv0, bv1, bv2, xwa, xwb, hf, outv,
          sem_a, sem_s, sem_o,
          *, n, f_in, fh, fo, nb):
    def slab(i):
        return pl.ds(i * _BM, _BM)

    a_cps = [pltpu.make_async_copy(a_ref.at[slab(i)], a32.at[slab(i)],
                                   sem_a.at[i]) for i in range(nb)]
    for cp in a_cps:
        cp.start()

    small = [
        pltpu.make_async_copy(x_ref, xv, sem_s.at[0]),
        pltpu.make_async_copy(w0_ref, wv0, sem_s.at[1]),
        pltpu.make_async_copy(b0_ref, bv0, sem_s.at[2]),
        pltpu.make_async_copy(w1_ref, wv1, sem_s.at[3]),
        pltpu.make_async_copy(b1_ref, bv1, sem_s.at[4]),
        pltpu.make_async_copy(w2_ref, wv2, sem_s.at[5]),
        pltpu.make_async_copy(b2_ref, bv2, sem_s.at[6]),
    ]
    for cp in small:
        cp.start()
    for cp in small:
        cp.wait()

    xwa[...] = jnp.dot(xv[...], wv0[...], preferred_element_type=jnp.float32)

    a_cps[0].wait()
    z_prev = jnp.dot(a32[slab(0)], xwa[...],
                     preferred_element_type=jnp.float32) + bv0[...]
    for i in range(1, nb):
        a_cps[i].wait()
        z = jnp.dot(a32[slab(i)], xwa[...],
                    preferred_element_type=jnp.float32) + bv0[...]
        hf[slab(i - 1)] = _norm_rows(z_prev, fh)
        xwb[slab(i - 1)] = jnp.dot(hf[slab(i - 1)], wv1[...],
                                   preferred_element_type=jnp.float32)
        z_prev = z
    hf[slab(nb - 1)] = _norm_rows(z_prev, fh)
    xwb[slab(nb - 1)] = jnp.dot(hf[slab(nb - 1)], wv1[...],
                                preferred_element_type=jnp.float32)

    z_prev = jnp.dot(a32[slab(0)], xwb[...],
                     preferred_element_type=jnp.float32) + bv1[...]
    for i in range(1, nb):
        z = jnp.dot(a32[slab(i)], xwb[...],
                    preferred_element_type=jnp.float32) + bv1[...]
        hf[slab(i - 1)] = _norm_rows(z_prev, fh)
        xwa[slab(i - 1)] = jnp.dot(hf[slab(i - 1)], wv2[...],
                                   preferred_element_type=jnp.float32)
        z_prev = z
    hf[slab(nb - 1)] = _norm_rows(z_prev, fh)
    xwa[slab(nb - 1)] = jnp.dot(hf[slab(nb - 1)], wv2[...],
                                preferred_element_type=jnp.float32)
    cp_pen = pltpu.make_async_copy(hf, pen_ref, sem_o.at[0])
    cp_pen.start()

    z_prev = jnp.dot(a32[slab(0)], xwa[...],
                     preferred_element_type=jnp.float32) + bv2[...]
    for i in range(1, nb):
        z = jnp.dot(a32[slab(i)], xwa[...],
                    preferred_element_type=jnp.float32) + bv2[...]
        outv[slab(i - 1)] = z_prev[:, :fo]
        z_prev = z
    outv[slab(nb - 1)] = z_prev[:, :fo]
    cp_out = pltpu.make_async_copy(outv, out_ref, sem_o.at[1])
    cp_out.start()
    cp_pen.wait()
    cp_out.wait()


def kernel(x, a_hat, W0, b0, W1, b1, W2, b2):
    n, f_in = x.shape
    fh = W0.shape[1]
    fo = W2.shape[1]
    nb = n // _BM

    w2 = jnp.zeros((fh, fh), jnp.float32).at[:, :fo].set(W2)
    b2p = jnp.zeros((1, fh), jnp.float32).at[:, :fo].set(b2.reshape(1, -1))

    body = functools.partial(_body, n=n, f_in=f_in, fh=fh, fo=fo, nb=nb)
    flops = 3 * 2 * n * n * fh + 2 * n * (f_in + 2 * fh) * fh
    out, pen = pl.pallas_call(
        body,
        out_shape=(jax.ShapeDtypeStruct((n, fo), jnp.float32),
                   jax.ShapeDtypeStruct((n, fh), jnp.float32)),
        in_specs=[pl.BlockSpec(memory_space=pl.ANY)] * 8,
        out_specs=(pl.BlockSpec(memory_space=pl.ANY),
                   pl.BlockSpec(memory_space=pl.ANY)),
        scratch_shapes=[
            pltpu.VMEM((n, n), jnp.float32),
            pltpu.VMEM((n, f_in), jnp.float32),
            pltpu.VMEM((f_in, fh), jnp.float32),
            pltpu.VMEM((fh, fh), jnp.float32),
            pltpu.VMEM((fh, fh), jnp.float32),
            pltpu.VMEM((1, fh), jnp.float32),
            pltpu.VMEM((1, fh), jnp.float32),
            pltpu.VMEM((1, fh), jnp.float32),
            pltpu.VMEM((n, fh), jnp.float32),
            pltpu.VMEM((n, fh), jnp.float32),
            pltpu.VMEM((n, fh), jnp.float32),
            pltpu.VMEM((n, fo), jnp.float32),
            pltpu.SemaphoreType.DMA((nb,)),
            pltpu.SemaphoreType.DMA((7,)),
            pltpu.SemaphoreType.DMA((2,)),
        ],
        compiler_params=pltpu.CompilerParams(
            vmem_limit_bytes=52 * 1024 * 1024,
        ),
        cost_estimate=pl.CostEstimate(
            flops=flops,
            transcendentals=2 * n,
            bytes_accessed=4 * (n * n + 4 * n * fh),
        ),
    )(x, a_hat, W0, b0.reshape(1, -1), W1, b1.reshape(1, -1), w2, b2p)
    return out, pen
